# Initial kernel scaffold; baseline (speedup 1.0000x reference)
import jax
import jax.numpy as jnp
from jax.experimental import pallas as pl


def kernel(input_labels, pos_labels, neg_labels, in_table, out_table):
    raise NotImplementedError("write your pallas kernel here")


# speedup vs baseline: 1.3494x; 1.3494x over previous
import functools

import jax
import jax.numpy as jnp
from jax import lax
from jax.experimental import pallas as pl
from jax.experimental.pallas import tpu as pltpu
from jax.experimental.pallas import tpu_sc as plsc

VOCAB = 1000000
EMBED = 64
BATCH = 16384
POS = 10
NEG = 50

NC = 2
NS = 16
NW = NC * NS

CH = 128

N_IN = BATCH
N_POS = BATCH * POS
N_NEG = BATCH * NEG


def _gather_range(table, idx_vmem, out_hbm, rows_v, sem, base, nch):
    def step(g, carry):
        off = base + g * CH
        pltpu.async_copy(
            table.at[idx_vmem.at[pl.ds(g * CH, CH)]], rows_v, sem
        ).wait()
        pltpu.sync_copy(rows_v, out_hbm.at[pl.ds(off, CH)])
        return carry

    lax.fori_loop(0, nch, step, None)


def _body(in_table, out_table, in_lbl, pos_lbl, neg_lbl,
          out_in, out_pos, out_neg,
          idx_in_v, idx_pos_v, idx_neg_v, rows_v, sem):
    wid = lax.axis_index("s") * NC + lax.axis_index("c")

    in_pw = N_IN // NW
    pos_pw = N_POS // NW
    neg_pw = N_NEG // NW

    pltpu.sync_copy(in_lbl.at[pl.ds(wid * in_pw, in_pw)], idx_in_v)
    pltpu.sync_copy(pos_lbl.at[pl.ds(wid * pos_pw, pos_pw)], idx_pos_v)
    pltpu.sync_copy(neg_lbl.at[pl.ds(wid * neg_pw, neg_pw)], idx_neg_v)

    _gather_range(in_table, idx_in_v, out_in, rows_v, sem,
                  wid * in_pw, in_pw // CH)
    _gather_range(out_table, idx_pos_v, out_pos, rows_v, sem,
                  wid * pos_pw, pos_pw // CH)
    _gather_range(out_table, idx_neg_v, out_neg, rows_v, sem,
                  wid * neg_pw, neg_pw // CH)


@jax.jit
def _run(in_table, out_table, in_lbl, pos_lbl, neg_lbl):
    mesh = plsc.VectorSubcoreMesh(core_axis_name="c", subcore_axis_name="s")
    f = pl.kernel(
        _body,
        out_type=[
            jax.ShapeDtypeStruct((N_IN, EMBED), jnp.float32),
            jax.ShapeDtypeStruct((N_POS, EMBED), jnp.float32),
            jax.ShapeDtypeStruct((N_NEG, EMBED), jnp.float32),
        ],
        mesh=mesh,
        compiler_params=pltpu.CompilerParams(use_tc_tiling_on_sc=False),
        scratch_types=[
            pltpu.VMEM((N_IN // NW,), jnp.int32),
            pltpu.VMEM((N_POS // NW,), jnp.int32),
            pltpu.VMEM((N_NEG // NW,), jnp.int32),
            pltpu.VMEM((CH, EMBED), jnp.float32),
            pltpu.SemaphoreType.DMA,
        ],
    )
    return f(in_table, out_table, in_lbl, pos_lbl, neg_lbl)


def kernel(input_labels, pos_labels, neg_labels, in_table, out_table):
    in_lbl = input_labels.astype(jnp.int32)
    pos_flat = pos_labels.astype(jnp.int32).reshape(-1)
    neg_flat = neg_labels.astype(jnp.int32).reshape(-1)
    out_in, out_pos, out_neg = _run(in_table, out_table,
                                    in_lbl, pos_flat, neg_flat)
    return (out_in,
            out_pos.reshape(BATCH, POS, EMBED),
            out_neg.reshape(BATCH, NEG, EMBED))


# speedup vs baseline: 1.4476x; 1.0728x over previous
import functools

import jax
import jax.numpy as jnp
from jax import lax
from jax.experimental import pallas as pl
from jax.experimental.pallas import tpu as pltpu
from jax.experimental.pallas import tpu_sc as plsc

VOCAB = 1000000
EMBED = 64
BATCH = 16384
POS = 10
NEG = 50

NC = 2
NS = 16
NW = NC * NS

CH = 512

N_IN = BATCH
N_POS = BATCH * POS
N_NEG = BATCH * NEG


def _gather_range(table, idx_vmem, out_hbm, rows_v, sem, base, nch):
    def step(g, carry):
        off = base + g * CH
        pltpu.async_copy(
            table.at[idx_vmem.at[pl.ds(g * CH, CH)]], rows_v, sem
        ).wait()
        pltpu.sync_copy(rows_v, out_hbm.at[pl.ds(off, CH)])
        return carry

    lax.fori_loop(0, nch, step, None)


def _body(in_table, out_table, in_lbl, pos_lbl, neg_lbl,
          out_in, out_pos, out_neg,
          idx_in_v, idx_pos_v, idx_neg_v, rows_v, sem):
    wid = lax.axis_index("s") * NC + lax.axis_index("c")

    in_pw = N_IN // NW
    pos_pw = N_POS // NW
    neg_pw = N_NEG // NW

    pltpu.sync_copy(in_lbl.at[pl.ds(wid * in_pw, in_pw)], idx_in_v)
    pltpu.sync_copy(pos_lbl.at[pl.ds(wid * pos_pw, pos_pw)], idx_pos_v)
    pltpu.sync_copy(neg_lbl.at[pl.ds(wid * neg_pw, neg_pw)], idx_neg_v)

    _gather_range(in_table, idx_in_v, out_in, rows_v, sem,
                  wid * in_pw, in_pw // CH)
    _gather_range(out_table, idx_pos_v, out_pos, rows_v, sem,
                  wid * pos_pw, pos_pw // CH)
    _gather_range(out_table, idx_neg_v, out_neg, rows_v, sem,
                  wid * neg_pw, neg_pw // CH)


@jax.jit
def _run(in_table, out_table, in_lbl, pos_lbl, neg_lbl):
    mesh = plsc.VectorSubcoreMesh(core_axis_name="c", subcore_axis_name="s")
    f = pl.kernel(
        _body,
        out_type=[
            jax.ShapeDtypeStruct((N_IN, EMBED), jnp.float32),
            jax.ShapeDtypeStruct((N_POS, EMBED), jnp.float32),
            jax.ShapeDtypeStruct((N_NEG, EMBED), jnp.float32),
        ],
        mesh=mesh,
        compiler_params=pltpu.CompilerParams(use_tc_tiling_on_sc=False),
        scratch_types=[
            pltpu.VMEM((N_IN // NW,), jnp.int32),
            pltpu.VMEM((N_POS // NW,), jnp.int32),
            pltpu.VMEM((N_NEG // NW,), jnp.int32),
            pltpu.VMEM((CH, EMBED), jnp.float32),
            pltpu.SemaphoreType.DMA,
        ],
    )
    return f(in_table, out_table, in_lbl, pos_lbl, neg_lbl)


def kernel(input_labels, pos_labels, neg_labels, in_table, out_table):
    in_lbl = input_labels.astype(jnp.int32)
    pos_flat = pos_labels.astype(jnp.int32).reshape(-1)
    neg_flat = neg_labels.astype(jnp.int32).reshape(-1)
    out_in, out_pos, out_neg = _run(in_table, out_table,
                                    in_lbl, pos_flat, neg_flat)
    return (out_in,
            out_pos.reshape(BATCH, POS, EMBED),
            out_neg.reshape(BATCH, NEG, EMBED))


# speedup vs baseline: 1.4724x; 1.0171x over previous
import functools

import jax
import jax.numpy as jnp
from jax import lax
from jax.experimental import pallas as pl
from jax.experimental.pallas import tpu as pltpu
from jax.experimental.pallas import tpu_sc as plsc

VOCAB = 1000000
EMBED = 64
BATCH = 16384
POS = 10
NEG = 50

NC = 2
NS = 16
NW = NC * NS

CH = 512

N_IN = BATCH
N_POS = BATCH * POS
N_NEG = BATCH * NEG


def _gather_desc(table, idx_vmem, rows_v, sem, c):
    return pltpu.make_async_copy(
        table.at[idx_vmem.at[pl.ds(c * CH, CH)]], rows_v, sem)


def _store_desc(out_hbm, rows_v, sem, base, c):
    return pltpu.make_async_copy(
        rows_v, out_hbm.at[pl.ds(base + c * CH, CH)], sem)


def _gather_range(table, idx_vmem, out_hbm, rows_a, rows_b,
                  gsa, gsb, ssa, ssb, base, nch):
    if nch == 1:
        _gather_desc(table, idx_vmem, rows_a, gsa, 0).start()
        _gather_desc(table, idx_vmem, rows_a, gsa, 0).wait()
        pltpu.sync_copy(rows_a, out_hbm.at[pl.ds(base, CH)])
        return

    _gather_desc(table, idx_vmem, rows_a, gsa, 0).start()

    def pair(p, carry):
        g = 2 * p
        _gather_desc(table, idx_vmem, rows_b, gsb, g + 1).start()
        _gather_desc(table, idx_vmem, rows_a, gsa, g).wait()
        _store_desc(out_hbm, rows_a, ssa, base, g).start()

        @pl.when(g + 2 < nch)
        def _():
            _store_desc(out_hbm, rows_a, ssa, base, g).wait()
            _gather_desc(table, idx_vmem, rows_a, gsa, g + 2).start()

        _gather_desc(table, idx_vmem, rows_b, gsb, g + 1).wait()
        _store_desc(out_hbm, rows_b, ssb, base, g + 1).start()
        _store_desc(out_hbm, rows_b, ssb, base, g + 1).wait()
        return carry

    lax.fori_loop(0, nch // 2, pair, None)
    _store_desc(out_hbm, rows_a, ssa, base, nch - 2).wait()


def _body(in_table, out_table, in_lbl, pos_lbl, neg_lbl,
          out_in, out_pos, out_neg,
          idx_in_v, idx_pos_v, idx_neg_v, rows_a, rows_b,
          gsa, gsb, ssa, ssb):
    wid = lax.axis_index("s") * NC + lax.axis_index("c")

    in_pw = N_IN // NW
    pos_pw = N_POS // NW
    neg_pw = N_NEG // NW

    pltpu.sync_copy(in_lbl.at[pl.ds(wid * in_pw, in_pw)], idx_in_v)
    pltpu.sync_copy(pos_lbl.at[pl.ds(wid * pos_pw, pos_pw)], idx_pos_v)
    pltpu.sync_copy(neg_lbl.at[pl.ds(wid * neg_pw, neg_pw)], idx_neg_v)

    _gather_range(in_table, idx_in_v, out_in, rows_a, rows_b,
                  gsa, gsb, ssa, ssb, wid * in_pw, in_pw // CH)
    _gather_range(out_table, idx_pos_v, out_pos, rows_a, rows_b,
                  gsa, gsb, ssa, ssb, wid * pos_pw, pos_pw // CH)
    _gather_range(out_table, idx_neg_v, out_neg, rows_a, rows_b,
                  gsa, gsb, ssa, ssb, wid * neg_pw, neg_pw // CH)


@jax.jit
def _run(in_table, out_table, in_lbl, pos_lbl, neg_lbl):
    mesh = plsc.VectorSubcoreMesh(core_axis_name="c", subcore_axis_name="s")
    f = pl.kernel(
        _body,
        out_type=[
            jax.ShapeDtypeStruct((N_IN, EMBED), jnp.float32),
            jax.ShapeDtypeStruct((N_POS, EMBED), jnp.float32),
            jax.ShapeDtypeStruct((N_NEG, EMBED), jnp.float32),
        ],
        mesh=mesh,
        compiler_params=pltpu.CompilerParams(use_tc_tiling_on_sc=False),
        scratch_types=[
            pltpu.VMEM((N_IN // NW,), jnp.int32),
            pltpu.VMEM((N_POS // NW,), jnp.int32),
            pltpu.VMEM((N_NEG // NW,), jnp.int32),
            pltpu.VMEM((CH, EMBED), jnp.float32),
            pltpu.VMEM((CH, EMBED), jnp.float32),
            pltpu.SemaphoreType.DMA,
            pltpu.SemaphoreType.DMA,
            pltpu.SemaphoreType.DMA,
            pltpu.SemaphoreType.DMA,
        ],
    )
    return f(in_table, out_table, in_lbl, pos_lbl, neg_lbl)


def kernel(input_labels, pos_labels, neg_labels, in_table, out_table):
    in_lbl = input_labels.astype(jnp.int32)
    pos_flat = pos_labels.astype(jnp.int32).reshape(-1)
    neg_flat = neg_labels.astype(jnp.int32).reshape(-1)
    out_in, out_pos, out_neg = _run(in_table, out_table,
                                    in_lbl, pos_flat, neg_flat)
    return (out_in,
            out_pos.reshape(BATCH, POS, EMBED),
            out_neg.reshape(BATCH, NEG, EMBED))
